# baseline (speedup 1.0000x reference)
import numpy as np

import jax
import jax.numpy as jnp
from jax.experimental import pallas as pl
from jax.experimental.pallas import tpu as pltpu


def _pick_tb(b, slab_bytes, budget):
    for d in range(b, 0, -1):
        if b % d == 0 and d * slab_bytes <= budget:
            return d
    return 1


def _mlp_softmax(y, w1t_ref, w2t_ref):
    hcur = jnp.dot(y, w1t_ref[...], preferred_element_type=jnp.float32)
    hcur = jnp.maximum(hcur, 0.0)
    logits = jnp.dot(hcur, w2t_ref[...], preferred_element_type=jnp.float32)
    m = jnp.max(logits, axis=-1, keepdims=True)
    e = jnp.exp(logits - m)
    return e * pl.reciprocal(jnp.sum(e, axis=-1, keepdims=True), approx=False)


def _se_pair_path(x, w1, w2):
    b, c, h, w = x.shape
    hid = w1.shape[0]
    out_ch = w2.shape[0]
    hw = h * w
    npair = c // 2
    qrows = 2 * hw // 128
    qf = hw // 128
    rem = hw % 128

    xv = x.reshape(b, npair, qrows, 128)

    q_pad = -(-qrows // 8) * 8
    slab_bytes = npair * q_pad * 128 * 4
    tb = _pick_tb(b, slab_bytes, 15 << 20)
    nb = b // tb

    def _body(x_ref, w1t_ref, w2t_ref, o_ref):
        x4 = x_ref[...]
        ya = jnp.sum(x4[:, :, :qf, :], axis=2)
        yb = jnp.sum(x4[:, :, qf + 1:, :], axis=2)
        mid = x4[:, :, qf, :]
        lane = jax.lax.broadcasted_iota(jnp.int32, mid.shape, 2)
        y_even = jnp.sum(ya + jnp.where(lane < rem, mid, 0.0), axis=-1)
        y_odd = jnp.sum(yb + jnp.where(lane >= rem, mid, 0.0), axis=-1)
        y = jnp.concatenate([y_even, y_odd], axis=1)
        o_ref[...] = _mlp_softmax(y, w1t_ref, w2t_ref)[None]

    perm = np.concatenate([np.arange(0, c, 2), np.arange(1, c, 2)])
    w1tp = (jnp.asarray(w1).T * (1.0 / hw))[perm, :]
    w2t = jnp.asarray(w2).T

    vmem_limit = min(2 * tb * slab_bytes + (4 << 20), 56 << 20)

    out = pl.pallas_call(
        _body,
        out_shape=jax.ShapeDtypeStruct((nb, tb, out_ch), jnp.float32),
        grid=(nb,),
        in_specs=[
            pl.BlockSpec((tb, npair, qrows, 128), lambda i: (i, 0, 0, 0)),
            pl.BlockSpec((c, hid), lambda i: (0, 0)),
            pl.BlockSpec((hid, out_ch), lambda i: (0, 0)),
        ],
        out_specs=pl.BlockSpec((1, tb, out_ch), lambda i: (i, 0, 0)),
        compiler_params=pltpu.CompilerParams(
            dimension_semantics=("parallel",),
            vmem_limit_bytes=vmem_limit,
        ),
    )(xv, w1tp, w2t)

    return out.reshape(b, out_ch, 1, 1)


def _se_flat_path(x, w1, w2):
    b, c, h, w = x.shape
    hid = w1.shape[0]
    out_ch = w2.shape[0]
    hw = h * w
    x_flat = x.reshape(b, c, hw)

    hw_pad = -(-hw // 128) * 128
    slab_bytes = c * hw_pad * 4
    tb = _pick_tb(b, slab_bytes, 15 << 20)
    nb = b // tb

    def _body(x_ref, w1t_ref, w2t_ref, o_ref):
        y = jnp.sum(x_ref[...], axis=-1)
        o_ref[...] = _mlp_softmax(y, w1t_ref, w2t_ref)[None]

    w1t = jnp.asarray(w1).T * (1.0 / hw)
    w2t = jnp.asarray(w2).T

    vmem_limit = min(2 * tb * slab_bytes + (4 << 20), 56 << 20)

    out = pl.pallas_call(
        _body,
        out_shape=jax.ShapeDtypeStruct((nb, tb, out_ch), jnp.float32),
        grid=(nb,),
        in_specs=[
            pl.BlockSpec((tb, c, hw), lambda i: (i, 0, 0)),
            pl.BlockSpec((c, hid), lambda i: (0, 0)),
            pl.BlockSpec((hid, out_ch), lambda i: (0, 0)),
        ],
        out_specs=pl.BlockSpec((1, tb, out_ch), lambda i: (i, 0, 0)),
        compiler_params=pltpu.CompilerParams(
            dimension_semantics=("parallel",),
            vmem_limit_bytes=vmem_limit,
        ),
    )(x_flat, w1t, w2t)

    return out.reshape(b, out_ch, 1, 1)


def kernel(x, w1, w2):
    b, c, h, w = x.shape
    hw = h * w
    if c % 2 == 0 and (2 * hw) % 128 == 0 and hw > 128:
        return _se_pair_path(x, w1, w2)
    return _se_flat_path(x, w1, w2)


# speedup vs baseline: 2.7235x; 2.7235x over previous
import jax
import jax.numpy as jnp
from jax.experimental import pallas as pl
from jax.experimental.pallas import tpu as pltpu


def _pick_tb(b, slab_bytes, budget):
    for d in range(b, 0, -1):
        if b % d == 0 and d * slab_bytes <= budget:
            return d
    return 1


def _se_layer(x, w1, w2):
    b, c, h, w = x.shape
    hid, c_in = w1.shape
    out_ch, hid2 = w2.shape
    assert c_in == c and hid2 == hid

    hw = h * w
    x_flat = x.reshape(b, c, hw)
    inv_hw = 1.0 / hw

    hw_pad = -(-hw // 128) * 128
    slab_bytes = c * hw_pad * 4
    tb = _pick_tb(b, slab_bytes, 15 << 20)
    nb = b // tb

    def _body(x_ref, w1_ref, w2_ref, o_ref):
        y = jnp.sum(x_ref[...], axis=-1) * inv_hw
        hcur = jax.lax.dot_general(
            y, w1_ref[...], (((1,), (1,)), ((), ())),
            preferred_element_type=jnp.float32)
        hcur = jnp.maximum(hcur, 0.0)
        logits = jax.lax.dot_general(
            hcur, w2_ref[...], (((1,), (1,)), ((), ())),
            preferred_element_type=jnp.float32)
        m = jnp.max(logits, axis=-1, keepdims=True)
        e = jnp.exp(logits - m)
        probs = e * pl.reciprocal(jnp.sum(e, axis=-1, keepdims=True),
                                  approx=False)
        o_ref[...] = probs[:, :, None, None]

    vmem_limit = min(2 * tb * slab_bytes + (4 << 20), 56 << 20)

    out = pl.pallas_call(
        _body,
        out_shape=jax.ShapeDtypeStruct((b, out_ch, 1, 1), jnp.float32),
        grid=(nb,),
        in_specs=[
            pl.BlockSpec((tb, c, hw), lambda i: (i, 0, 0)),
            pl.BlockSpec((hid, c), lambda i: (0, 0)),
            pl.BlockSpec((out_ch, hid), lambda i: (0, 0)),
        ],
        out_specs=pl.BlockSpec((tb, out_ch, 1, 1), lambda i: (i, 0, 0, 0)),
        compiler_params=pltpu.CompilerParams(
            dimension_semantics=("parallel",),
            vmem_limit_bytes=vmem_limit,
        ),
    )(x_flat, w1, w2)

    return out


def kernel(x, w1, w2):
    return _se_layer(x, w1, w2)


# speedup vs baseline: 2.9265x; 1.0746x over previous
import jax
import jax.numpy as jnp
from jax.experimental import pallas as pl
from jax.experimental.pallas import tpu as pltpu


def _pick_tb(b, slab_bytes, budget):
    for d in range(b, 0, -1):
        if b % d == 0 and d * slab_bytes <= budget:
            return d
    return 1


def _se_layer(x, w1, w2):
    b, c, h, w = x.shape
    hid, c_in = w1.shape
    out_ch, hid2 = w2.shape
    assert c_in == c and hid2 == hid

    hw = h * w
    x_flat = x.reshape(b, c, hw)
    inv_hw = 1.0 / hw

    hw_pad = -(-hw // 128) * 128
    slab_bytes = c * hw_pad * 4
    tb = _pick_tb(b, slab_bytes, 15 << 20)
    nb = b // tb

    def _body(x_ref, w1_ref, w2_ref, o_ref):
        y = jnp.sum(x_ref[...], axis=-1) * inv_hw
        hcur = jax.lax.dot_general(
            y, w1_ref[...], (((1,), (1,)), ((), ())),
            preferred_element_type=jnp.float32)
        hcur = jnp.maximum(hcur, 0.0)
        logits = jax.lax.dot_general(
            hcur, w2_ref[...], (((1,), (1,)), ((), ())),
            preferred_element_type=jnp.float32)
        m = jnp.max(logits, axis=-1, keepdims=True)
        e = jnp.exp(logits - m)
        probs = e * pl.reciprocal(jnp.sum(e, axis=-1, keepdims=True),
                                  approx=False)
        o_ref[...] = probs[None]

    vmem_limit = min(2 * tb * slab_bytes + (4 << 20), 56 << 20)

    out = pl.pallas_call(
        _body,
        out_shape=jax.ShapeDtypeStruct((nb, tb, out_ch), jnp.float32),
        grid=(nb,),
        in_specs=[
            pl.BlockSpec((tb, c, hw), lambda i: (i, 0, 0)),
            pl.BlockSpec((hid, c), lambda i: (0, 0)),
            pl.BlockSpec((out_ch, hid), lambda i: (0, 0)),
        ],
        out_specs=pl.BlockSpec((1, tb, out_ch), lambda i: (i, 0, 0)),
        compiler_params=pltpu.CompilerParams(
            dimension_semantics=("parallel",),
            vmem_limit_bytes=vmem_limit,
        ),
    )(x_flat, w1, w2)

    return out.reshape(b, out_ch, 1, 1)


def kernel(x, w1, w2):
    return _se_layer(x, w1, w2)


# speedup vs baseline: 11.9819x; 4.0942x over previous
import jax
import jax.numpy as jnp
from jax.experimental import pallas as pl
from jax.experimental.pallas import tpu as pltpu


def _pick_tb(b, slab_bytes, budget):
    for d in range(b, 0, -1):
        if b % d == 0 and d * slab_bytes <= budget:
            return d
    return 1


def _se_layer(x, w1, w2):
    b, c, h, w = x.shape
    hid, c_in = w1.shape
    out_ch, hid2 = w2.shape
    assert c_in == c and hid2 == hid

    hw = h * w
    xt = jnp.transpose(x.reshape(b, c, hw), (0, 2, 1))
    inv_hw = 1.0 / hw

    c_pad = -(-c // 128) * 128
    hw_s = -(-hw // 8) * 8
    slab_bytes = hw_s * c_pad * 4
    tb = _pick_tb(b, slab_bytes, 15 << 20)
    nb = b // tb

    def _body(x_ref, w1_ref, w2_ref, o_ref):
        y = jnp.sum(x_ref[...], axis=1) * inv_hw
        hcur = jax.lax.dot_general(
            y, w1_ref[...], (((1,), (1,)), ((), ())),
            preferred_element_type=jnp.float32)
        hcur = jnp.maximum(hcur, 0.0)
        logits = jax.lax.dot_general(
            hcur, w2_ref[...], (((1,), (1,)), ((), ())),
            preferred_element_type=jnp.float32)
        m = jnp.max(logits, axis=-1, keepdims=True)
        e = jnp.exp(logits - m)
        probs = e * pl.reciprocal(jnp.sum(e, axis=-1, keepdims=True),
                                  approx=False)
        o_ref[...] = probs[None]

    vmem_limit = min(2 * tb * slab_bytes + (4 << 20), 56 << 20)

    out = pl.pallas_call(
        _body,
        out_shape=jax.ShapeDtypeStruct((nb, tb, out_ch), jnp.float32),
        grid=(nb,),
        in_specs=[
            pl.BlockSpec((tb, hw, c), lambda i: (i, 0, 0)),
            pl.BlockSpec((hid, c), lambda i: (0, 0)),
            pl.BlockSpec((out_ch, hid), lambda i: (0, 0)),
        ],
        out_specs=pl.BlockSpec((1, tb, out_ch), lambda i: (i, 0, 0)),
        compiler_params=pltpu.CompilerParams(
            dimension_semantics=("parallel",),
            vmem_limit_bytes=vmem_limit,
        ),
    )(xt, w1, w2)

    return out.reshape(b, out_ch, 1, 1)


def kernel(x, w1, w2):
    return _se_layer(x, w1, w2)
